# baseline (speedup 1.0000x reference)
import functools

import jax
import jax.numpy as jnp
from jax import lax
from jax.experimental import pallas as pl
from jax.experimental.pallas import tpu as pltpu
from jax.experimental.pallas import tpu_sc as plsc

_NC = 2
_NS = 16
_L = 16
_NW = _NC * _NS

_BLK = 64
_CP = 256


def _vsplat(vec, i):
    idx = jnp.full((_L,), i, jnp.int32)
    return lax.gather(
        vec, idx[:, None],
        lax.GatherDimensionNumbers(offset_dims=(), collapsed_slice_dims=(0,),
                                   start_index_map=(0,)),
        (1,), mode=lax.GatherScatterMode.PROMISE_IN_BOUNDS)


def _build_table(features_b, eye):
    _, C, H, W = features_b.shape
    HB = 8

    def body(f_ref, eye_ref, o_ref):
        ey = eye_ref[...]
        for i in range(HB):
            a = f_ref[0, :, i, :]
            o_ref[0, i] = lax.dot_general(
                a, ey, (((0,), (0,)), ((), ())),
                preferred_element_type=jnp.float32)

    return pl.pallas_call(
        body,
        grid=(1, H // HB),
        in_specs=[
            pl.BlockSpec((1, C, HB, W), lambda b, h: (b, 0, h, 0)),
            pl.BlockSpec((C, _CP), lambda b, h: (0, 0)),
        ],
        out_specs=pl.BlockSpec((1, HB, W, _CP), lambda b, h: (b, h, 0, 0)),
        out_shape=jax.ShapeDtypeStruct((1, H, W, _CP), jnp.float32),
    )(features_b, eye)


def _make_sampler(B, C, H, W, N):
    TOTAL = B * N
    P_PER_W = TOTAL // _NW
    NBLK = P_PER_W // _BLK
    HW = H * W
    NCG = C // _L

    mesh = plsc.VectorSubcoreMesh(core_axis_name="c", subcore_axis_name="s",
                                  num_cores=_NC, num_subcores=_NS)

    @functools.partial(
        pl.kernel,
        out_type=jax.ShapeDtypeStruct((TOTAL * C,), jnp.float32),
        mesh=mesh,
        scratch_types=[
            pltpu.VMEM((P_PER_W,), jnp.float32),
            pltpu.VMEM((P_PER_W,), jnp.float32),
            pltpu.VMEM((_BLK,), jnp.int32),
            pltpu.VMEM((_BLK,), jnp.int32),
            pltpu.VMEM((_BLK,), jnp.int32),
            pltpu.VMEM((_BLK,), jnp.int32),
            pltpu.VMEM((_BLK,), jnp.float32),
            pltpu.VMEM((_BLK,), jnp.float32),
            pltpu.VMEM((_BLK,), jnp.float32),
            pltpu.VMEM((_BLK,), jnp.float32),
            pltpu.VMEM((_BLK, _CP), jnp.float32),
            pltpu.VMEM((_BLK, _CP), jnp.float32),
            pltpu.VMEM((_BLK, _CP), jnp.float32),
            pltpu.VMEM((_BLK, _CP), jnp.float32),
            pltpu.VMEM((_BLK * C,), jnp.float32),
            pltpu.SemaphoreType.DMA,
        ],
        compiler_params=pltpu.CompilerParams(use_tc_tiling_on_sc=True),
    )
    def sampler(table, xs_hbm, ys_hbm, out_hbm,
                xs_v, ys_v, i00, i01, i10, i11,
                wa_v, wb_v, wc_v, wd_v,
                r00, r01, r10, r11, ob, sem):
        wid = lax.axis_index("s") * _NC + lax.axis_index("c")
        base = wid * P_PER_W
        pltpu.sync_copy(xs_hbm.at[pl.ds(base, P_PER_W)], xs_v)
        pltpu.sync_copy(ys_hbm.at[pl.ds(base, P_PER_W)], ys_v)
        boff = (base // N) * HW

        def block(blk, carry):
            pbase = blk * _BLK
            for g in range(_BLK // _L):
                s = pl.ds(pbase + g * _L, _L)
                d = pl.ds(g * _L, _L)
                vx = (2.0 * xs_v[s] - 1.0 + 1.0) * 0.5 * (W - 1)
                vy = (2.0 * ys_v[s] - 1.0 + 1.0) * 0.5 * (H - 1)
                xi = jnp.maximum(vx, 0.0).astype(jnp.int32)
                yi = jnp.maximum(vy, 0.0).astype(jnp.int32)
                fx = vx - xi.astype(jnp.float32)
                fy = vy - yi.astype(jnp.float32)
                x0 = jnp.minimum(xi, W - 1)
                x1 = jnp.minimum(xi + 1, W - 1)
                y0 = jnp.minimum(yi, H - 1)
                y1 = jnp.minimum(yi + 1, H - 1)
                r0 = boff + y0 * W
                r1 = boff + y1 * W
                i00[d] = r0 + x0
                i01[d] = r1 + x0
                i10[d] = r0 + x1
                i11[d] = r1 + x1
                wa_v[d] = (1.0 - fx) * (1.0 - fy)
                wb_v[d] = (1.0 - fx) * fy
                wc_v[d] = fx * (1.0 - fy)
                wd_v[d] = fx * fy
            c0 = pltpu.async_copy(table.at[i00], r00, sem)
            c1 = pltpu.async_copy(table.at[i01], r01, sem)
            c2 = pltpu.async_copy(table.at[i10], r10, sem)
            c3 = pltpu.async_copy(table.at[i11], r11, sem)
            c0.wait()
            c1.wait()
            c2.wait()
            c3.wait()

            for g in range(_BLK // _L):
                d = pl.ds(g * _L, _L)
                wa_g = wa_v[d]
                wb_g = wb_v[d]
                wc_g = wc_v[d]
                wd_g = wd_v[d]

                def point(p, pc, wa_g=wa_g, wb_g=wb_g, wc_g=wc_g,
                          wd_g=wd_g, g=g):
                    wa_s = _vsplat(wa_g, p)
                    wb_s = _vsplat(wb_g, p)
                    wc_s = _vsplat(wc_g, p)
                    wd_s = _vsplat(wd_g, p)
                    row = g * _L + p
                    for cg in range(NCG):
                        cs = pl.ds(cg * _L, _L)
                        acc = (r00[row, cs] * wa_s + r01[row, cs] * wb_s
                               + r10[row, cs] * wc_s + r11[row, cs] * wd_s)
                        ob[pl.ds(row * C + cg * _L, _L)] = acc
                    return pc

                lax.fori_loop(0, _L, point, 0)
            pltpu.sync_copy(ob, out_hbm.at[pl.ds((base + pbase) * C, _BLK * C)])
            return carry

        lax.fori_loop(0, NBLK, block, 0)

    return sampler


def kernel(features, points):
    B, C, H, W = features.shape
    N = points.shape[1]
    eye = jnp.eye(C, _CP, dtype=jnp.float32)
    sampler = _make_sampler(1, C, H, W, N)
    outs = []
    for b in range(B):
        table = _build_table(features[b:b + 1], eye).reshape(H * W, _CP)
        outs.append(sampler(table, points[b, :, 0], points[b, :, 1]))
    return jnp.stack(outs).reshape(B, N, C)


# speedup vs baseline: 1.4266x; 1.4266x over previous
import functools

import jax
import jax.numpy as jnp
from jax import lax
from jax.experimental import pallas as pl
from jax.experimental.pallas import tpu as pltpu
from jax.experimental.pallas import tpu_sc as plsc

_NC = 2
_NS = 16
_L = 16
_NW = _NC * _NS

_BLK = 32
_CP = 256


def _vsplat(vec, i):
    idx = jnp.full((_L,), i, jnp.int32)
    return lax.gather(
        vec, idx[:, None],
        lax.GatherDimensionNumbers(offset_dims=(), collapsed_slice_dims=(0,),
                                   start_index_map=(0,)),
        (1,), mode=lax.GatherScatterMode.PROMISE_IN_BOUNDS)


def _build_table(features):
    B, C, H, W = features.shape
    HB = 8
    eye = jnp.eye(C, _CP, dtype=jnp.float32)

    def body(f_ref, eye_ref, o_ref):
        ey = eye_ref[...]
        for i in range(HB):
            a = f_ref[0, :, i, :]
            o_ref[0, i] = lax.dot_general(
                a, ey, (((0,), (0,)), ((), ())),
                preferred_element_type=jnp.float32)

    return pl.pallas_call(
        body,
        grid=(B, H // HB),
        in_specs=[
            pl.BlockSpec((1, C, HB, W), lambda b, h: (b, 0, h, 0)),
            pl.BlockSpec((C, _CP), lambda b, h: (0, 0)),
        ],
        out_specs=pl.BlockSpec((1, HB, W, _CP), lambda b, h: (b, h, 0, 0)),
        out_shape=jax.ShapeDtypeStruct((B, H, W, _CP), jnp.float32),
    )(features, eye)


def _make_sampler(B, C, H, W, N):
    TOTAL = B * N
    P_PER_W = TOTAL // _NW
    NBLK = P_PER_W // _BLK
    HW = H * W
    NCG = C // _L
    NG = _BLK // _L

    mesh = plsc.VectorSubcoreMesh(core_axis_name="c", subcore_axis_name="s",
                                  num_cores=_NC, num_subcores=_NS)

    buf = lambda shape, dt: [pltpu.VMEM(shape, dt) for _ in range(2)]

    @functools.partial(
        pl.kernel,
        out_type=jax.ShapeDtypeStruct((TOTAL * C,), jnp.float32),
        mesh=mesh,
        scratch_types=[
            pltpu.VMEM((P_PER_W,), jnp.float32),
            pltpu.VMEM((P_PER_W,), jnp.float32),
            buf((_BLK,), jnp.int32),
            buf((_BLK,), jnp.int32),
            buf((_BLK,), jnp.int32),
            buf((_BLK,), jnp.int32),
            buf((_BLK,), jnp.float32),
            buf((_BLK,), jnp.float32),
            buf((_BLK,), jnp.float32),
            buf((_BLK,), jnp.float32),
            buf((_BLK, _CP), jnp.float32),
            buf((_BLK, _CP), jnp.float32),
            buf((_BLK, _CP), jnp.float32),
            buf((_BLK, _CP), jnp.float32),
            pltpu.VMEM((_BLK * C,), jnp.float32),
            pltpu.SemaphoreType.DMA,
        ],
        compiler_params=pltpu.CompilerParams(use_tc_tiling_on_sc=True),
    )
    def sampler(table, xs_hbm, ys_hbm, out_hbm,
                xs_v, ys_v, i00, i01, i10, i11,
                wa_v, wb_v, wc_v, wd_v,
                r00, r01, r10, r11, ob, sem):
        wid = lax.axis_index("s") * _NC + lax.axis_index("c")
        base = wid * P_PER_W
        pltpu.sync_copy(xs_hbm.at[pl.ds(base, P_PER_W)], xs_v)
        pltpu.sync_copy(ys_hbm.at[pl.ds(base, P_PER_W)], ys_v)
        boff = (base // N) * HW

        def stage(blk, j):
            pbase = blk * _BLK
            for g in range(NG):
                s = pl.ds(pbase + g * _L, _L)
                d = pl.ds(g * _L, _L)
                vx = (2.0 * xs_v[s] - 1.0 + 1.0) * 0.5 * (W - 1)
                vy = (2.0 * ys_v[s] - 1.0 + 1.0) * 0.5 * (H - 1)
                xi = jnp.maximum(vx, 0.0).astype(jnp.int32)
                yi = jnp.maximum(vy, 0.0).astype(jnp.int32)
                fx = vx - xi.astype(jnp.float32)
                fy = vy - yi.astype(jnp.float32)
                x0 = jnp.minimum(xi, W - 1)
                x1 = jnp.minimum(xi + 1, W - 1)
                y0 = jnp.minimum(yi, H - 1)
                y1 = jnp.minimum(yi + 1, H - 1)
                r0 = boff + y0 * W
                r1 = boff + y1 * W
                i00[j][d] = r0 + x0
                i01[j][d] = r1 + x0
                i10[j][d] = r0 + x1
                i11[j][d] = r1 + x1
                wa_v[j][d] = (1.0 - fx) * (1.0 - fy)
                wb_v[j][d] = (1.0 - fx) * fy
                wc_v[j][d] = fx * (1.0 - fy)
                wd_v[j][d] = fx * fy
            pltpu.async_copy(table.at[i00[j]], r00[j], sem)
            pltpu.async_copy(table.at[i01[j]], r01[j], sem)
            pltpu.async_copy(table.at[i10[j]], r10[j], sem)
            pltpu.async_copy(table.at[i11[j]], r11[j], sem)

        def drain(j):
            pltpu.make_async_copy(table.at[i00[j]], r00[j], sem).wait()
            pltpu.make_async_copy(table.at[i01[j]], r01[j], sem).wait()
            pltpu.make_async_copy(table.at[i10[j]], r10[j], sem).wait()
            pltpu.make_async_copy(table.at[i11[j]], r11[j], sem).wait()

        def combine(blk, j):
            pbase = blk * _BLK
            for g in range(NG):
                d = pl.ds(g * _L, _L)
                wa_g = wa_v[j][d]
                wb_g = wb_v[j][d]
                wc_g = wc_v[j][d]
                wd_g = wd_v[j][d]

                def point(p, pc, wa_g=wa_g, wb_g=wb_g, wc_g=wc_g,
                          wd_g=wd_g, g=g):
                    wa_s = _vsplat(wa_g, p)
                    wb_s = _vsplat(wb_g, p)
                    wc_s = _vsplat(wc_g, p)
                    wd_s = _vsplat(wd_g, p)
                    row = g * _L + p
                    for cg in range(NCG):
                        cs = pl.ds(cg * _L, _L)
                        acc = (r00[j][row, cs] * wa_s + r01[j][row, cs] * wb_s
                               + r10[j][row, cs] * wc_s + r11[j][row, cs] * wd_s)
                        ob[pl.ds(row * C + cg * _L, _L)] = acc
                    return pc

                lax.fori_loop(0, _L, point, 0)
            pltpu.sync_copy(ob, out_hbm.at[pl.ds((base + pbase) * C, _BLK * C)])

        stage(0, 0)

        def pair(k, carry):
            for j in range(2):
                blk = 2 * k + j
                nxt = blk + 1

                @pl.when(nxt < NBLK)
                def _():
                    stage(nxt, 1 - j)

                drain(j)
                combine(blk, j)
            return carry

        lax.fori_loop(0, NBLK // 2, pair, 0)

    return sampler


def kernel(features, points):
    B, C, H, W = features.shape
    N = points.shape[1]
    table = _build_table(features).reshape(B * H * W, _CP)
    xs = points[:, :, 0].reshape(B * N)
    ys = points[:, :, 1].reshape(B * N)
    sampler = _make_sampler(B, C, H, W, N)
    out = sampler(table, xs, ys)
    return out.reshape(B, N, C)


# speedup vs baseline: 1.5559x; 1.0906x over previous
import functools

import jax
import jax.numpy as jnp
from jax import lax
from jax.experimental import pallas as pl
from jax.experimental.pallas import tpu as pltpu
from jax.experimental.pallas import tpu_sc as plsc

_NC = 2
_NS = 16
_L = 16
_NW = _NC * _NS

_BLK = 32
_CP = 256


def _vsplat(vec, i):
    idx = jnp.full((_L,), i, jnp.int32)
    return lax.gather(
        vec, idx[:, None],
        lax.GatherDimensionNumbers(offset_dims=(), collapsed_slice_dims=(0,),
                                   start_index_map=(0,)),
        (1,), mode=lax.GatherScatterMode.PROMISE_IN_BOUNDS)


def _build_table(features):
    B, C, H, W = features.shape
    HB = 16
    eye = jnp.eye(C, _CP, dtype=jnp.float32)

    def body(f_ref, eye_ref, o_ref):
        ey = eye_ref[...]
        for i in range(HB):
            a = f_ref[0, :, i, :]
            o_ref[0, i] = lax.dot_general(
                a, ey, (((0,), (0,)), ((), ())),
                preferred_element_type=jnp.float32)

    return pl.pallas_call(
        body,
        grid=(B, H // HB),
        in_specs=[
            pl.BlockSpec((1, C, HB, W), lambda b, h: (b, 0, h, 0)),
            pl.BlockSpec((C, _CP), lambda b, h: (0, 0)),
        ],
        out_specs=pl.BlockSpec((1, HB, W, _CP), lambda b, h: (b, h, 0, 0)),
        out_shape=jax.ShapeDtypeStruct((B, H, W, _CP), jnp.float32),
    )(features, eye)


def _make_sampler(B, C, H, W, N):
    TOTAL = B * N
    P_PER_W = TOTAL // _NW
    NBLK = P_PER_W // _BLK
    HW = H * W
    NCG = C // _L
    NG = _BLK // _L

    mesh = plsc.VectorSubcoreMesh(core_axis_name="c", subcore_axis_name="s",
                                  num_cores=_NC, num_subcores=_NS)

    buf = lambda shape, dt: [pltpu.VMEM(shape, dt) for _ in range(2)]

    @functools.partial(
        pl.kernel,
        out_type=jax.ShapeDtypeStruct((TOTAL * C,), jnp.float32),
        mesh=mesh,
        scratch_types=[
            pltpu.VMEM((P_PER_W,), jnp.float32),
            pltpu.VMEM((P_PER_W,), jnp.float32),
            buf((_BLK,), jnp.int32),
            buf((_BLK,), jnp.int32),
            buf((_BLK,), jnp.int32),
            buf((_BLK,), jnp.int32),
            buf((_BLK,), jnp.float32),
            buf((_BLK,), jnp.float32),
            buf((_BLK,), jnp.float32),
            buf((_BLK,), jnp.float32),
            buf((_BLK, _CP), jnp.float32),
            buf((_BLK, _CP), jnp.float32),
            buf((_BLK, _CP), jnp.float32),
            buf((_BLK, _CP), jnp.float32),
            pltpu.VMEM((_BLK * C,), jnp.float32),
            pltpu.SemaphoreType.DMA,
        ],
        compiler_params=pltpu.CompilerParams(use_tc_tiling_on_sc=True),
    )
    def sampler(table, xs_hbm, ys_hbm, out_hbm,
                xs_v, ys_v, i00, i01, i10, i11,
                wa_v, wb_v, wc_v, wd_v,
                r00, r01, r10, r11, ob, sem):
        wid = lax.axis_index("s") * _NC + lax.axis_index("c")
        base = wid * P_PER_W
        pltpu.sync_copy(xs_hbm.at[pl.ds(base, P_PER_W)], xs_v)
        pltpu.sync_copy(ys_hbm.at[pl.ds(base, P_PER_W)], ys_v)
        boff = (base // N) * HW

        def stage(blk, j):
            pbase = blk * _BLK
            for g in range(NG):
                s = pl.ds(pbase + g * _L, _L)
                d = pl.ds(g * _L, _L)
                vx = (2.0 * xs_v[s] - 1.0 + 1.0) * 0.5 * (W - 1)
                vy = (2.0 * ys_v[s] - 1.0 + 1.0) * 0.5 * (H - 1)
                xi = jnp.maximum(vx, 0.0).astype(jnp.int32)
                yi = jnp.maximum(vy, 0.0).astype(jnp.int32)
                fx = vx - xi.astype(jnp.float32)
                fy = vy - yi.astype(jnp.float32)
                x0 = jnp.minimum(xi, W - 1)
                x1 = jnp.minimum(xi + 1, W - 1)
                y0 = jnp.minimum(yi, H - 1)
                y1 = jnp.minimum(yi + 1, H - 1)
                r0 = boff + y0 * W
                r1 = boff + y1 * W
                i00[j][d] = r0 + x0
                i01[j][d] = r1 + x0
                i10[j][d] = r0 + x1
                i11[j][d] = r1 + x1
                wa_v[j][d] = (1.0 - fx) * (1.0 - fy)
                wb_v[j][d] = (1.0 - fx) * fy
                wc_v[j][d] = fx * (1.0 - fy)
                wd_v[j][d] = fx * fy
            pltpu.async_copy(table.at[i00[j]], r00[j], sem)
            pltpu.async_copy(table.at[i01[j]], r01[j], sem)
            pltpu.async_copy(table.at[i10[j]], r10[j], sem)
            pltpu.async_copy(table.at[i11[j]], r11[j], sem)

        def drain(j):
            pltpu.make_async_copy(table.at[i00[j]], r00[j], sem).wait()
            pltpu.make_async_copy(table.at[i01[j]], r01[j], sem).wait()
            pltpu.make_async_copy(table.at[i10[j]], r10[j], sem).wait()
            pltpu.make_async_copy(table.at[i11[j]], r11[j], sem).wait()

        def combine(blk, j):
            pbase = blk * _BLK
            for g in range(NG):
                d = pl.ds(g * _L, _L)
                wa_g = wa_v[j][d]
                wb_g = wb_v[j][d]
                wc_g = wc_v[j][d]
                wd_g = wd_v[j][d]

                def point(p, pc, wa_g=wa_g, wb_g=wb_g, wc_g=wc_g,
                          wd_g=wd_g, g=g):
                    wa_s = _vsplat(wa_g, p)
                    wb_s = _vsplat(wb_g, p)
                    wc_s = _vsplat(wc_g, p)
                    wd_s = _vsplat(wd_g, p)
                    row = g * _L + p
                    for cg in range(NCG):
                        cs = pl.ds(cg * _L, _L)
                        acc = (r00[j][row, cs] * wa_s + r01[j][row, cs] * wb_s
                               + r10[j][row, cs] * wc_s + r11[j][row, cs] * wd_s)
                        ob[pl.ds(row * C + cg * _L, _L)] = acc
                    return pc

                lax.fori_loop(0, _L, point, 0)
            pltpu.sync_copy(ob, out_hbm.at[pl.ds((base + pbase) * C, _BLK * C)])

        stage(0, 0)

        def pair(k, carry):
            for j in range(2):
                blk = 2 * k + j
                nxt = blk + 1

                @pl.when(nxt < NBLK)
                def _():
                    stage(nxt, 1 - j)

                drain(j)
                combine(blk, j)
            return carry

        lax.fori_loop(0, NBLK // 2, pair, 0)

    return sampler


def kernel(features, points):
    B, C, H, W = features.shape
    N = points.shape[1]
    table = _build_table(features).reshape(B * H * W, _CP)
    xs = points[:, :, 0].reshape(B * N)
    ys = points[:, :, 1].reshape(B * N)
    sampler = _make_sampler(B, C, H, W, N)
    out = sampler(table, xs, ys)
    return out.reshape(B, N, C)


# speedup vs baseline: 1.5851x; 1.0188x over previous
import functools

import jax
import jax.numpy as jnp
from jax import lax
from jax.experimental import pallas as pl
from jax.experimental.pallas import tpu as pltpu
from jax.experimental.pallas import tpu_sc as plsc

_NC = 2
_NS = 16
_L = 16
_NW = _NC * _NS

_BLK = 32
_CP = 256


def _vsplat(vec, i):
    idx = jnp.full((_L,), i, jnp.int32)
    return lax.gather(
        vec, idx[:, None],
        lax.GatherDimensionNumbers(offset_dims=(), collapsed_slice_dims=(0,),
                                   start_index_map=(0,)),
        (1,), mode=lax.GatherScatterMode.PROMISE_IN_BOUNDS)


def _build_table(features):
    B, C, H, W = features.shape
    HB = 32
    eye = jnp.eye(C, _CP, dtype=jnp.float32)

    def body(f_ref, eye_ref, o_ref):
        ey = eye_ref[...]
        for i in range(HB):
            a = f_ref[0, :, i, :]
            o_ref[0, i] = lax.dot_general(
                a, ey, (((0,), (0,)), ((), ())),
                preferred_element_type=jnp.float32)

    return pl.pallas_call(
        body,
        grid=(B, H // HB),
        in_specs=[
            pl.BlockSpec((1, C, HB, W), lambda b, h: (b, 0, h, 0)),
            pl.BlockSpec((C, _CP), lambda b, h: (0, 0)),
        ],
        out_specs=pl.BlockSpec((1, HB, W, _CP), lambda b, h: (b, h, 0, 0)),
        out_shape=jax.ShapeDtypeStruct((B, H, W, _CP), jnp.float32),
    )(features, eye)


def _make_sampler(B, C, H, W, N):
    TOTAL = B * N
    P_PER_W = TOTAL // _NW
    NBLK = P_PER_W // _BLK
    HW = H * W
    NCG = C // _L
    NG = _BLK // _L

    mesh = plsc.VectorSubcoreMesh(core_axis_name="c", subcore_axis_name="s",
                                  num_cores=_NC, num_subcores=_NS)

    buf = lambda shape, dt: [pltpu.VMEM(shape, dt) for _ in range(2)]

    @functools.partial(
        pl.kernel,
        out_type=jax.ShapeDtypeStruct((TOTAL * C,), jnp.float32),
        mesh=mesh,
        scratch_types=[
            pltpu.VMEM((P_PER_W,), jnp.float32),
            pltpu.VMEM((P_PER_W,), jnp.float32),
            buf((_BLK,), jnp.int32),
            buf((_BLK,), jnp.int32),
            buf((_BLK,), jnp.int32),
            buf((_BLK,), jnp.int32),
            buf((_BLK,), jnp.float32),
            buf((_BLK,), jnp.float32),
            buf((_BLK,), jnp.float32),
            buf((_BLK,), jnp.float32),
            buf((_BLK, _CP), jnp.float32),
            buf((_BLK, _CP), jnp.float32),
            buf((_BLK, _CP), jnp.float32),
            buf((_BLK, _CP), jnp.float32),
            pltpu.VMEM((_BLK * C,), jnp.float32),
            pltpu.SemaphoreType.DMA,
        ],
        compiler_params=pltpu.CompilerParams(use_tc_tiling_on_sc=True),
    )
    def sampler(table, xs_hbm, ys_hbm, out_hbm,
                xs_v, ys_v, i00, i01, i10, i11,
                wa_v, wb_v, wc_v, wd_v,
                r00, r01, r10, r11, ob, sem):
        wid = lax.axis_index("s") * _NC + lax.axis_index("c")
        base = wid * P_PER_W
        pltpu.sync_copy(xs_hbm.at[pl.ds(base, P_PER_W)], xs_v)
        pltpu.sync_copy(ys_hbm.at[pl.ds(base, P_PER_W)], ys_v)
        boff = (base // N) * HW

        def stage(blk, j):
            pbase = blk * _BLK
            for g in range(NG):
                s = pl.ds(pbase + g * _L, _L)
                d = pl.ds(g * _L, _L)
                vx = (2.0 * xs_v[s] - 1.0 + 1.0) * 0.5 * (W - 1)
                vy = (2.0 * ys_v[s] - 1.0 + 1.0) * 0.5 * (H - 1)
                xi = jnp.maximum(vx, 0.0).astype(jnp.int32)
                yi = jnp.maximum(vy, 0.0).astype(jnp.int32)
                fx = vx - xi.astype(jnp.float32)
                fy = vy - yi.astype(jnp.float32)
                x0 = jnp.minimum(xi, W - 1)
                x1 = jnp.minimum(xi + 1, W - 1)
                y0 = jnp.minimum(yi, H - 1)
                y1 = jnp.minimum(yi + 1, H - 1)
                r0 = boff + y0 * W
                r1 = boff + y1 * W
                i00[j][d] = r0 + x0
                i01[j][d] = r1 + x0
                i10[j][d] = r0 + x1
                i11[j][d] = r1 + x1
                wa_v[j][d] = (1.0 - fx) * (1.0 - fy)
                wb_v[j][d] = (1.0 - fx) * fy
                wc_v[j][d] = fx * (1.0 - fy)
                wd_v[j][d] = fx * fy
            pltpu.async_copy(table.at[i00[j]], r00[j], sem)
            pltpu.async_copy(table.at[i01[j]], r01[j], sem)
            pltpu.async_copy(table.at[i10[j]], r10[j], sem)
            pltpu.async_copy(table.at[i11[j]], r11[j], sem)

        def drain(j):
            pltpu.make_async_copy(table.at[i00[j]], r00[j], sem).wait()
            pltpu.make_async_copy(table.at[i01[j]], r01[j], sem).wait()
            pltpu.make_async_copy(table.at[i10[j]], r10[j], sem).wait()
            pltpu.make_async_copy(table.at[i11[j]], r11[j], sem).wait()

        def combine(blk, j):
            pbase = blk * _BLK
            for g in range(NG):
                d = pl.ds(g * _L, _L)
                wa_g = wa_v[j][d]
                wb_g = wb_v[j][d]
                wc_g = wc_v[j][d]
                wd_g = wd_v[j][d]

                def point(p, pc, wa_g=wa_g, wb_g=wb_g, wc_g=wc_g,
                          wd_g=wd_g, g=g):
                    wa_s = _vsplat(wa_g, p)
                    wb_s = _vsplat(wb_g, p)
                    wc_s = _vsplat(wc_g, p)
                    wd_s = _vsplat(wd_g, p)
                    row = g * _L + p
                    for cg in range(NCG):
                        cs = pl.ds(cg * _L, _L)
                        acc = (r00[j][row, cs] * wa_s + r01[j][row, cs] * wb_s
                               + r10[j][row, cs] * wc_s + r11[j][row, cs] * wd_s)
                        ob[pl.ds(row * C + cg * _L, _L)] = acc
                    return pc

                lax.fori_loop(0, _L, point, 0)
            pltpu.sync_copy(ob, out_hbm.at[pl.ds((base + pbase) * C, _BLK * C)])

        stage(0, 0)

        def pair(k, carry):
            for j in range(2):
                blk = 2 * k + j
                nxt = blk + 1

                @pl.when(nxt < NBLK)
                def _():
                    stage(nxt, 1 - j)

                drain(j)
                combine(blk, j)
            return carry

        lax.fori_loop(0, NBLK // 2, pair, 0)

    return sampler


def kernel(features, points):
    B, C, H, W = features.shape
    N = points.shape[1]
    table = _build_table(features).reshape(B * H * W, _CP)
    xs = points[:, :, 0].reshape(B * N)
    ys = points[:, :, 1].reshape(B * N)
    sampler = _make_sampler(B, C, H, W, N)
    out = sampler(table, xs, ys)
    return out.reshape(B, N, C)
